# baseline (device time: 1068958 ns/iter reference)
import jax
import jax.numpy as jnp
from jax import lax
from jax.experimental import pallas as pl
from jax.experimental.pallas import tpu as pltpu

M_PER = 8192
N_PER = 1024
HALF = M_PER // 2
NC = 16
R = HALF // NC


def kernel(x):
    def body(
        x_ref,
        out_ref,
        stage,
        local_sem,
        load_sems,
        send_sems_x,
        recv_sems_x,
        fwd_send_sems,
        recv_sems_y,
    ):
        my_x = lax.axis_index("x")
        my_y = lax.axis_index("y")
        peer_x = 1 - my_x

        local = pltpu.make_async_copy(
            x_ref.at[:, pl.ds(my_x * N_PER, N_PER)],
            out_ref.at[pl.ds(my_x * M_PER, M_PER), :],
            local_sem,
        )
        local.start()

        send_base = my_y * HALF
        xland_base = my_x * M_PER + my_y * HALF
        xrecv_base = peer_x * M_PER + my_y * HALF
        yrecv_base = peer_x * M_PER + (1 - my_y) * HALF

        def load(k):
            return pltpu.make_async_copy(
                x_ref.at[pl.ds(send_base + k * R, R), pl.ds(peer_x * N_PER, N_PER)],
                stage.at[k % 2],
                load_sems.at[k % 2],
            )

        def x_send(k):
            return pltpu.make_async_remote_copy(
                src_ref=stage.at[k % 2],
                dst_ref=out_ref.at[pl.ds(xland_base + k * R, R), :],
                send_sem=send_sems_x.at[k % 2],
                recv_sem=recv_sems_x.at[k],
                device_id=(peer_x, my_y),
                device_id_type=pl.DeviceIdType.MESH,
            )

        def x_recv(k):
            return pltpu.make_async_remote_copy(
                src_ref=stage.at[k % 2],
                dst_ref=out_ref.at[pl.ds(xrecv_base + k * R, R), :],
                send_sem=send_sems_x.at[k % 2],
                recv_sem=recv_sems_x.at[k],
                device_id=(peer_x, my_y),
                device_id_type=pl.DeviceIdType.MESH,
            )

        def fwd(k):
            return pltpu.make_async_remote_copy(
                src_ref=out_ref.at[pl.ds(xrecv_base + k * R, R), :],
                dst_ref=out_ref.at[pl.ds(xrecv_base + k * R, R), :],
                send_sem=fwd_send_sems.at[k % 2],
                recv_sem=recv_sems_y.at[k],
                device_id=(my_x, 1 - my_y),
                device_id_type=pl.DeviceIdType.MESH,
            )

        def y_recv(k):
            return pltpu.make_async_remote_copy(
                src_ref=out_ref.at[pl.ds(yrecv_base + k * R, R), :],
                dst_ref=out_ref.at[pl.ds(yrecv_base + k * R, R), :],
                send_sem=fwd_send_sems.at[k % 2],
                recv_sem=recv_sems_y.at[k],
                device_id=(my_x, 1 - my_y),
                device_id_type=pl.DeviceIdType.MESH,
            )

        loads = [load(k) for k in range(NC)]
        xsends = [x_send(k) for k in range(NC)]
        xrecvs = [x_recv(k) for k in range(NC)]
        fwds = [fwd(k) for k in range(NC)]
        yrecvs = [y_recv(k) for k in range(NC)]

        loads[0].start()
        for k in range(NC):
            loads[k].wait()
            xsends[k].start()
            if k + 1 < NC:
                if k >= 1:
                    xsends[k - 1].wait_send()
                loads[k + 1].start()
            xrecvs[k].wait_recv()
            if k >= 2:
                fwds[k - 2].wait_send()
            fwds[k].start()
        for k in range(NC):
            yrecvs[k].wait_recv()
        xsends[NC - 2].wait_send()
        xsends[NC - 1].wait_send()
        fwds[NC - 2].wait_send()
        fwds[NC - 1].wait_send()
        local.wait()

    return pl.pallas_call(
        body,
        out_shape=jax.ShapeDtypeStruct((2 * M_PER, N_PER), jnp.float32),
        in_specs=[pl.BlockSpec(memory_space=pl.ANY)],
        out_specs=pl.BlockSpec(memory_space=pl.ANY),
        scratch_shapes=[
            pltpu.VMEM((2, R, N_PER), jnp.float32),
            pltpu.SemaphoreType.DMA,
            pltpu.SemaphoreType.DMA((2,)),
            pltpu.SemaphoreType.DMA((2,)),
            pltpu.SemaphoreType.DMA((NC,)),
            pltpu.SemaphoreType.DMA((2,)),
            pltpu.SemaphoreType.DMA((NC,)),
        ],
    )(x)


# device time: 274235 ns/iter; 3.8980x vs baseline; 3.8980x over previous
import jax
import jax.numpy as jnp
from jax import lax
from jax.experimental import pallas as pl
from jax.experimental.pallas import tpu as pltpu

M_PER = 8192
N_PER = 1024
HALF = M_PER // 2
NCH = 8
R = HALF // NCH
NT = 8
TL = M_PER // NT


def kernel(x):
    def body(
        x_ref,
        out_ref,
        lbuf,
        sbuf,
        rbuf,
        lload_sems,
        lstore_sems,
        sload_sems,
        xsend_sems,
        xrecv_sems,
        wr_sems,
        fwd_send_sems,
        yrecv_sems,
        credit_sem,
    ):
        my_x = lax.axis_index("x")
        my_y = lax.axis_index("y")
        peer_x = 1 - my_x

        send_base = my_y * HALF
        xland_base = my_x * M_PER + my_y * HALF
        xrecv_base = peer_x * M_PER + my_y * HALF
        yrecv_base = peer_x * M_PER + (1 - my_y) * HALF

        lloads = [
            pltpu.make_async_copy(
                x_ref.at[pl.ds(t * TL, TL), pl.ds(my_x * N_PER, N_PER)],
                lbuf.at[t % 2],
                lload_sems.at[t % 2],
            )
            for t in range(NT)
        ]
        lstores = [
            pltpu.make_async_copy(
                lbuf.at[t % 2],
                out_ref.at[pl.ds(my_x * M_PER + t * TL, TL), :],
                lstore_sems.at[t % 2],
            )
            for t in range(NT)
        ]
        sloads = [
            pltpu.make_async_copy(
                x_ref.at[pl.ds(send_base + k * R, R), pl.ds(peer_x * N_PER, N_PER)],
                sbuf.at[k % 2],
                sload_sems.at[k % 2],
            )
            for k in range(NCH)
        ]
        xsends = [
            pltpu.make_async_remote_copy(
                src_ref=sbuf.at[k % 2],
                dst_ref=rbuf.at[k % 2],
                send_sem=xsend_sems.at[k % 2],
                recv_sem=xrecv_sems.at[k],
                device_id=(peer_x, my_y),
                device_id_type=pl.DeviceIdType.MESH,
            )
            for k in range(NCH)
        ]
        wrs = [
            pltpu.make_async_copy(
                rbuf.at[k % 2],
                out_ref.at[pl.ds(xrecv_base + k * R, R), :],
                wr_sems.at[k % 2],
            )
            for k in range(NCH)
        ]
        fwds = [
            pltpu.make_async_remote_copy(
                src_ref=rbuf.at[k % 2],
                dst_ref=out_ref.at[pl.ds(xrecv_base + k * R, R), :],
                send_sem=fwd_send_sems.at[k % 2],
                recv_sem=yrecv_sems.at[k],
                device_id=(my_x, 1 - my_y),
                device_id_type=pl.DeviceIdType.MESH,
            )
            for k in range(NCH)
        ]
        yrecvs = [
            pltpu.make_async_remote_copy(
                src_ref=rbuf.at[k % 2],
                dst_ref=out_ref.at[pl.ds(yrecv_base + k * R, R), :],
                send_sem=fwd_send_sems.at[k % 2],
                recv_sem=yrecv_sems.at[k],
                device_id=(my_x, 1 - my_y),
                device_id_type=pl.DeviceIdType.MESH,
            )
            for k in range(NCH)
        ]

        lloads[0].start()
        sloads[0].start()
        for k in range(NCH):
            sloads[k].wait()
            if k >= 2:
                pl.semaphore_wait(credit_sem, 1)
            xsends[k].start()
            if k + 1 < NCH:
                if k >= 1:
                    xsends[k - 1].wait_send()
                sloads[k + 1].start()
            lloads[k].wait()
            lstores[k].start()
            if k + 1 < NT:
                if k >= 1:
                    lstores[k - 1].wait()
                lloads[k + 1].start()
            xsends[k].wait_recv()
            wrs[k].start()
            fwds[k].start()
            if k >= 1:
                wrs[k - 1].wait()
                fwds[k - 1].wait_send()
                if k - 1 <= NCH - 3:
                    pl.semaphore_signal(
                        credit_sem,
                        inc=1,
                        device_id=(peer_x, my_y),
                        device_id_type=pl.DeviceIdType.MESH,
                    )
        xsends[NCH - 2].wait_send()
        xsends[NCH - 1].wait_send()
        lstores[NT - 2].wait()
        lstores[NT - 1].wait()
        wrs[NCH - 1].wait()
        fwds[NCH - 1].wait_send()
        for k in range(NCH):
            yrecvs[k].wait_recv()

    return pl.pallas_call(
        body,
        out_shape=jax.ShapeDtypeStruct((2 * M_PER, N_PER), jnp.float32),
        in_specs=[pl.BlockSpec(memory_space=pl.ANY)],
        out_specs=pl.BlockSpec(memory_space=pl.ANY),
        scratch_shapes=[
            pltpu.VMEM((2, TL, N_PER), jnp.float32),
            pltpu.VMEM((2, R, N_PER), jnp.float32),
            pltpu.VMEM((2, R, N_PER), jnp.float32),
            pltpu.SemaphoreType.DMA((2,)),
            pltpu.SemaphoreType.DMA((2,)),
            pltpu.SemaphoreType.DMA((2,)),
            pltpu.SemaphoreType.DMA((2,)),
            pltpu.SemaphoreType.DMA((NCH,)),
            pltpu.SemaphoreType.DMA((2,)),
            pltpu.SemaphoreType.DMA((2,)),
            pltpu.SemaphoreType.DMA((NCH,)),
            pltpu.SemaphoreType.REGULAR,
        ],
    )(x)


# device time: 246156 ns/iter; 4.3426x vs baseline; 1.1141x over previous
import jax
import jax.numpy as jnp
from jax import lax
from jax.experimental import pallas as pl
from jax.experimental.pallas import tpu as pltpu

M_PER = 8192
N_PER = 1024
HALF = M_PER // 2
NCH = 64
R = HALF // NCH
NT = 8
TL = M_PER // NT


def kernel(x):
    def body(
        x_ref,
        out_ref,
        lbuf,
        lload_sems,
        lstore_sems,
        xsend_sems,
        xrecv_sems,
        fwd_send_sems,
        yrecv_sems,
    ):
        my_x = lax.axis_index("x")
        my_y = lax.axis_index("y")
        peer_x = 1 - my_x

        send_base = my_y * HALF
        xland_base = my_x * M_PER + my_y * HALF
        xrecv_base = peer_x * M_PER + my_y * HALF
        yrecv_base = peer_x * M_PER + (1 - my_y) * HALF

        lloads = [
            pltpu.make_async_copy(
                x_ref.at[pl.ds(t * TL, TL), pl.ds(my_x * N_PER, N_PER)],
                lbuf.at[t % 2],
                lload_sems.at[t % 2],
            )
            for t in range(NT)
        ]
        lstores = [
            pltpu.make_async_copy(
                lbuf.at[t % 2],
                out_ref.at[pl.ds(my_x * M_PER + t * TL, TL), :],
                lstore_sems.at[t % 2],
            )
            for t in range(NT)
        ]
        xsends = [
            pltpu.make_async_remote_copy(
                src_ref=x_ref.at[pl.ds(send_base + k * R, R), pl.ds(peer_x * N_PER, N_PER)],
                dst_ref=out_ref.at[pl.ds(xland_base + k * R, R), :],
                send_sem=xsend_sems.at[k],
                recv_sem=xrecv_sems.at[k],
                device_id=(peer_x, my_y),
                device_id_type=pl.DeviceIdType.MESH,
            )
            for k in range(NCH)
        ]
        xrecvs = [
            pltpu.make_async_remote_copy(
                src_ref=x_ref.at[pl.ds(send_base + k * R, R), pl.ds(peer_x * N_PER, N_PER)],
                dst_ref=out_ref.at[pl.ds(xrecv_base + k * R, R), :],
                send_sem=xsend_sems.at[k],
                recv_sem=xrecv_sems.at[k],
                device_id=(peer_x, my_y),
                device_id_type=pl.DeviceIdType.MESH,
            )
            for k in range(NCH)
        ]
        fwds = [
            pltpu.make_async_remote_copy(
                src_ref=out_ref.at[pl.ds(xrecv_base + k * R, R), :],
                dst_ref=out_ref.at[pl.ds(xrecv_base + k * R, R), :],
                send_sem=fwd_send_sems.at[k],
                recv_sem=yrecv_sems.at[k],
                device_id=(my_x, 1 - my_y),
                device_id_type=pl.DeviceIdType.MESH,
            )
            for k in range(NCH)
        ]
        yrecvs = [
            pltpu.make_async_remote_copy(
                src_ref=out_ref.at[pl.ds(yrecv_base + k * R, R), :],
                dst_ref=out_ref.at[pl.ds(yrecv_base + k * R, R), :],
                send_sem=fwd_send_sems.at[k],
                recv_sem=yrecv_sems.at[k],
                device_id=(my_x, 1 - my_y),
                device_id_type=pl.DeviceIdType.MESH,
            )
            for k in range(NCH)
        ]

        for k in range(NCH):
            xsends[k].start()
        lloads[0].start()
        for k in range(NCH):
            if k < NT:
                lloads[k].wait()
                lstores[k].start()
                if k + 1 < NT:
                    if k >= 1:
                        lstores[k - 1].wait()
                    lloads[k + 1].start()
            xrecvs[k].wait_recv()
            fwds[k].start()
        lstores[NT - 2].wait()
        lstores[NT - 1].wait()
        for k in range(NCH):
            xsends[k].wait_send()
            fwds[k].wait_send()
            yrecvs[k].wait_recv()

    return pl.pallas_call(
        body,
        out_shape=jax.ShapeDtypeStruct((2 * M_PER, N_PER), jnp.float32),
        in_specs=[pl.BlockSpec(memory_space=pl.ANY)],
        out_specs=pl.BlockSpec(memory_space=pl.ANY),
        scratch_shapes=[
            pltpu.VMEM((2, TL, N_PER), jnp.float32),
            pltpu.SemaphoreType.DMA((2,)),
            pltpu.SemaphoreType.DMA((2,)),
            pltpu.SemaphoreType.DMA((NCH,)),
            pltpu.SemaphoreType.DMA((NCH,)),
            pltpu.SemaphoreType.DMA((NCH,)),
            pltpu.SemaphoreType.DMA((NCH,)),
        ],
    )(x)


# device time: 240125 ns/iter; 4.4517x vs baseline; 1.0251x over previous
import jax
import jax.numpy as jnp
from jax import lax
from jax.experimental import pallas as pl
from jax.experimental.pallas import tpu as pltpu

M_PER = 8192
N_PER = 1024
HALF = M_PER // 2
NCH = 64
R = HALF // NCH
NT = 16
TL = M_PER // NT


def kernel(x):
    def body(
        x_ref,
        out_ref,
        lbuf,
        lload_sems,
        lstore_sems,
        xsend_sems,
        xrecv_sems,
        fwd_send_sems,
        yrecv_sems,
    ):
        my_x = lax.axis_index("x")
        my_y = lax.axis_index("y")
        peer_x = 1 - my_x

        send_base = my_y * HALF
        xland_base = my_x * M_PER + my_y * HALF
        xrecv_base = peer_x * M_PER + my_y * HALF
        yrecv_base = peer_x * M_PER + (1 - my_y) * HALF

        lloads = [
            pltpu.make_async_copy(
                x_ref.at[pl.ds(t * TL, TL), pl.ds(my_x * N_PER, N_PER)],
                lbuf.at[t % 2],
                lload_sems.at[t % 2],
            )
            for t in range(NT)
        ]
        lstores = [
            pltpu.make_async_copy(
                lbuf.at[t % 2],
                out_ref.at[pl.ds(my_x * M_PER + t * TL, TL), :],
                lstore_sems.at[t % 2],
            )
            for t in range(NT)
        ]
        xsends = [
            pltpu.make_async_remote_copy(
                src_ref=x_ref.at[pl.ds(send_base + k * R, R), pl.ds(peer_x * N_PER, N_PER)],
                dst_ref=out_ref.at[pl.ds(xland_base + k * R, R), :],
                send_sem=xsend_sems.at[k],
                recv_sem=xrecv_sems.at[k],
                device_id=(peer_x, my_y),
                device_id_type=pl.DeviceIdType.MESH,
            )
            for k in range(NCH)
        ]
        xrecvs = [
            pltpu.make_async_remote_copy(
                src_ref=x_ref.at[pl.ds(send_base + k * R, R), pl.ds(peer_x * N_PER, N_PER)],
                dst_ref=out_ref.at[pl.ds(xrecv_base + k * R, R), :],
                send_sem=xsend_sems.at[k],
                recv_sem=xrecv_sems.at[k],
                device_id=(peer_x, my_y),
                device_id_type=pl.DeviceIdType.MESH,
            )
            for k in range(NCH)
        ]
        fwds = [
            pltpu.make_async_remote_copy(
                src_ref=out_ref.at[pl.ds(xrecv_base + k * R, R), :],
                dst_ref=out_ref.at[pl.ds(xrecv_base + k * R, R), :],
                send_sem=fwd_send_sems.at[k],
                recv_sem=yrecv_sems.at[k],
                device_id=(my_x, 1 - my_y),
                device_id_type=pl.DeviceIdType.MESH,
            )
            for k in range(NCH)
        ]
        yrecvs = [
            pltpu.make_async_remote_copy(
                src_ref=out_ref.at[pl.ds(yrecv_base + k * R, R), :],
                dst_ref=out_ref.at[pl.ds(yrecv_base + k * R, R), :],
                send_sem=fwd_send_sems.at[k],
                recv_sem=yrecv_sems.at[k],
                device_id=(my_x, 1 - my_y),
                device_id_type=pl.DeviceIdType.MESH,
            )
            for k in range(NCH)
        ]

        for k in range(NCH):
            xsends[k].start()
        lloads[0].start()
        for k in range(NCH):
            if k < NT:
                lloads[k].wait()
                lstores[k].start()
                if k + 1 < NT:
                    if k >= 1:
                        lstores[k - 1].wait()
                    lloads[k + 1].start()
            xrecvs[k].wait_recv()
            fwds[k].start()
        lstores[NT - 2].wait()
        lstores[NT - 1].wait()
        for k in range(NCH):
            xsends[k].wait_send()
            fwds[k].wait_send()
            yrecvs[k].wait_recv()

    return pl.pallas_call(
        body,
        out_shape=jax.ShapeDtypeStruct((2 * M_PER, N_PER), jnp.float32),
        in_specs=[pl.BlockSpec(memory_space=pl.ANY)],
        out_specs=pl.BlockSpec(memory_space=pl.ANY),
        scratch_shapes=[
            pltpu.VMEM((2, TL, N_PER), jnp.float32),
            pltpu.SemaphoreType.DMA((2,)),
            pltpu.SemaphoreType.DMA((2,)),
            pltpu.SemaphoreType.DMA((NCH,)),
            pltpu.SemaphoreType.DMA((NCH,)),
            pltpu.SemaphoreType.DMA((NCH,)),
            pltpu.SemaphoreType.DMA((NCH,)),
        ],
    )(x)
